# baseline (speedup 1.0000x reference)
import functools
import math

import jax
import jax.numpy as jnp
from jax import lax
from jax.experimental import pallas as pl
from jax.experimental.pallas import tpu as pltpu
from jax.experimental.pallas import tpu_sc as plsc

_S = 256
_L = 16
_NW = 32


def _rank_body(at_ref, s_ref, rank_out, stone_out, dist_out, ang_out,
               dist_scr):
    at = at_ref[...]
    s = s_ref[...]
    bn = at.shape[1]

    ayr = at[1:2, :]
    axr = at[2:3, :]
    sy = s[:, 1:2]
    sx = s[:, 2:3]

    dy = sy - ayr
    dx = sx - axr
    d2 = dy * dy + dx * dx
    dist = jnp.sqrt(d2)
    raw = jnp.arctan2(-dy, dx) * (180.0 / math.pi)
    ang = jnp.where(raw > 0, raw, raw + 360.0)
    stone = jnp.broadcast_to(s[:, 0:1], (_S, bn))

    dist_scr[...] = dist
    elem = lax.broadcasted_iota(jnp.int32, (_S, bn), 0)

    def jcol(j, acc):
        kj = dist_scr[j][None, :]
        lt = kj < dist
        le = kj <= dist
        cond = lt | (le & (elem > j))
        return acc + jnp.where(cond, 1.0, 0.0)

    acc = lax.fori_loop(0, _S, jcol, jnp.zeros((_S, bn), jnp.float32))
    rank3 = acc.astype(jnp.int32) * 3

    mask = at[0:1, :] == 0.0
    rank_out[...] = rank3
    stone_out[...] = jnp.where(mask, stone, 0.0)
    dist_out[...] = jnp.where(mask, dist, 0.0)
    ang_out[...] = jnp.where(mask, ang, 0.0)


def _tc_rank_call(a, s):
    n = a.shape[0]
    bn = 128
    grid = n // bn
    at = jnp.pad(a.T, ((0, 5), (0, 0)))
    out_shape = [jax.ShapeDtypeStruct((_S, n), jnp.int32)] + [
        jax.ShapeDtypeStruct((_S, n), jnp.float32)] * 3
    return pl.pallas_call(
        _rank_body,
        grid=(grid,),
        in_specs=[
            pl.BlockSpec((8, bn), lambda i: (0, i)),
            pl.BlockSpec((_S, 3), lambda i: (0, 0)),
        ],
        out_specs=[pl.BlockSpec((_S, bn), lambda i: (0, i))] * 4,
        out_shape=out_shape,
        scratch_shapes=[pltpu.VMEM((_S, bn), jnp.float32)],
    )(at, s)


def _sc_scatter_call(rank3, stone_m, dist_m, ang_m):
    n = rank3.shape[0] // _S
    rows_w = n // _NW
    groups = rows_w // _L
    mesh = plsc.VectorSubcoreMesh(core_axis_name="c", subcore_axis_name="s")

    @functools.partial(
        pl.kernel,
        out_type=jax.ShapeDtypeStruct((n * 3 * _S,), jnp.float32),
        mesh=mesh,
        scratch_types=[
            pltpu.VMEM((_L * _S,), jnp.int32),
            pltpu.VMEM((_L * _S,), jnp.float32),
            pltpu.VMEM((_L * _S,), jnp.float32),
            pltpu.VMEM((_L * _S,), jnp.float32),
            pltpu.VMEM((_L * 3 * _S,), jnp.float32),
        ],
        compiler_params=pltpu.CompilerParams(needs_layout_passes=False),
    )
    def k(rank_h, stone_h, dist_h, ang_h, out_h, rank_v, stone_v, dist_v,
          ang_v, out_v):
        wid = lax.axis_index("s") * 2 + lax.axis_index("c")

        def group_body(g, _):
            base = (wid * rows_w + g * _L) * _S
            pltpu.sync_copy(rank_h.at[pl.ds(base, _L * _S)], rank_v)
            pltpu.sync_copy(stone_h.at[pl.ds(base, _L * _S)], stone_v)
            pltpu.sync_copy(dist_h.at[pl.ds(base, _L * _S)], dist_v)
            pltpu.sync_copy(ang_h.at[pl.ds(base, _L * _S)], ang_v)

            def row_body(r, _):
                rbase = r * (3 * _S)

                def chunk_body(c, _):
                    o = r * _S + c * _L
                    r3 = rank_v[pl.ds(o, _L)] + rbase
                    plsc.store_scatter(out_v, [r3],
                                       stone_v[pl.ds(o, _L)])
                    plsc.store_scatter(out_v, [r3 + 1],
                                       dist_v[pl.ds(o, _L)])
                    plsc.store_scatter(out_v, [r3 + 2],
                                       ang_v[pl.ds(o, _L)])
                    return 0

                lax.fori_loop(0, _S // _L, chunk_body, 0, unroll=4)
                return 0

            lax.fori_loop(0, _L, row_body, 0)
            pltpu.sync_copy(out_v, out_h.at[pl.ds(base * 3, _L * 3 * _S)])
            return 0

        lax.fori_loop(0, groups, group_body, 0)

    return k(rank3, stone_m, dist_m, ang_m)


@jax.jit
def kernel(all_coord_input, stone_coord_input):
    a = all_coord_input.astype(jnp.float32)
    s = stone_coord_input.astype(jnp.float32)
    n = a.shape[0]
    rank3, stone_m, dist_m, ang_m = _tc_rank_call(a, s)
    out = _sc_scatter_call(rank3.T.reshape(-1), stone_m.T.reshape(-1),
                           dist_m.T.reshape(-1), ang_m.T.reshape(-1))
    return out.reshape(n, _S, 3)
